# baseline (speedup 1.0000x reference)
import functools

import jax
import jax.numpy as jnp
from jax import lax
from jax.experimental import pallas as pl
from jax.experimental.pallas import tpu as pltpu
from jax.experimental.pallas import tpu_sc as plsc

_B, _C, _A, _W, _O, _L, _K = 1024, 100, 6, 5, 4, 256, 2
_NC, _NS = 2, 16
_NW = _NC * _NS
_RPW = (_B * _K) // _NW


def _sc_gather_rows(start_tab, end_tab, idx):
  mesh = plsc.VectorSubcoreMesh(core_axis_name="c", subcore_axis_name="s")

  @functools.partial(
      pl.kernel,
      mesh=mesh,
      out_type=[
          jax.ShapeDtypeStruct((_B * _K, _L), jnp.float32),
          jax.ShapeDtypeStruct((_B * _K, _L), jnp.float32),
      ],
      scratch_types=[
          pltpu.VMEM((_RPW,), jnp.int32),
          pltpu.VMEM((_RPW, _L), jnp.float32),
          pltpu.VMEM((_RPW, _L), jnp.float32),
          pltpu.SemaphoreType.DMA,
          pltpu.SemaphoreType.DMA,
      ],
  )
  def gather_k(s_hbm, e_hbm, idx_hbm, s_out, e_out, idx_v, s_v, e_v,
               sem_s, sem_e):
    wid = lax.axis_index("s") * _NC + lax.axis_index("c")
    base = wid * _RPW
    pltpu.sync_copy(idx_hbm.at[pl.ds(base, _RPW)], idx_v)
    cp_s = pltpu.async_copy(s_hbm.at[idx_v], s_v, sem_s)
    cp_e = pltpu.async_copy(e_hbm.at[idx_v], e_v, sem_e)
    cp_s.wait()
    cp_e.wait()
    pltpu.sync_copy(s_v, s_out.at[pl.ds(base, _RPW)])
    pltpu.sync_copy(e_v, e_out.at[pl.ds(base, _RPW)])

  return gather_k(start_tab, end_tab, idx)


def _softplus(x):
  return jnp.maximum(x, 0.0) + jnp.log1p(jnp.exp(-jnp.abs(x)))


def _masked_ce_sum(x, sel_mask, picked_mask):
  xm = jnp.where(sel_mask, x, -jnp.inf)
  m = jnp.max(xm, axis=1, keepdims=True)
  lse = m + jnp.log(jnp.sum(jnp.exp(xm - m), axis=1, keepdims=True))
  picked = jnp.sum(jnp.where(picked_mask, x, 0.0), axis=1, keepdims=True)
  return jnp.sum(lse - picked)


def _tc_loss_body(sel_ref, agg_ref, num_ref, col_ref, op_ref, srow_ref,
                  erow_ref, selt_ref, aggt_ref, numt_ref, colt_ref, opt_ref,
                  stt_ref, ett_ref, out_ref):
  sel = sel_ref[...]
  cidx = lax.broadcasted_iota(jnp.int32, (_B, _C), 1)
  m_sel = jnp.max(sel, axis=1, keepdims=True)
  lse_sel = m_sel + jnp.log(jnp.sum(jnp.exp(sel - m_sel), axis=1,
                                    keepdims=True))
  picked_sel = jnp.sum(jnp.where(cidx == selt_ref[...], sel, 0.0), axis=1,
                       keepdims=True)
  loss = jnp.sum(lse_sel - picked_sel) * (1.0 / _B)
  amax_col = jnp.min(jnp.where(sel == m_sel, cidx, _C), axis=1,
                     keepdims=True)

  agg = agg_ref[...]
  ja = lax.broadcasted_iota(jnp.int32, (_B, _C * _A), 1)
  sel_mask = (ja // _A) == amax_col
  picked_mask = ja == (amax_col * _A + aggt_ref[...])
  loss = loss + _masked_ce_sum(agg, sel_mask, picked_mask) * (1.0 / _B)

  num = num_ref[...]
  jw = lax.broadcasted_iota(jnp.int32, (_B, _W), 1)
  m_num = jnp.max(num, axis=1, keepdims=True)
  lse_num = m_num + jnp.log(jnp.sum(jnp.exp(num - m_num), axis=1,
                                    keepdims=True))
  picked_num = jnp.sum(jnp.where(jw == numt_ref[...], num, 0.0), axis=1,
                       keepdims=True)
  loss = loss + jnp.sum(lse_num - picked_num) * (1.0 / _B)

  colw = col_ref[...]
  t0 = colt_ref[..., 0:1]
  t1 = colt_ref[..., 1:2]
  h = (cidx == t0) | (cidx == t1)
  sp_pos = _softplus(colw)
  sp_neg = sp_pos - colw
  bce_sum = jnp.sum(jnp.where(h, 3.0 * sp_neg, sp_pos))
  loss = loss + bce_sum * (jnp.float32(_B) / _C)

  op = op_ref[...]
  jo = lax.broadcasted_iota(jnp.int32, (_B, _C * _O), 1)
  op_sum = jnp.float32(0.0)
  for k in range(_K):
    ck = colt_ref[..., k:k + 1]
    tk = opt_ref[..., k:k + 1]
    op_sum = op_sum + _masked_ce_sum(op, (jo // _O) == ck,
                                     jo == (ck * _O + tk))
  loss = loss + op_sum * (1.0 / (_B * _K))

  jl = lax.broadcasted_iota(jnp.int32, (_B * _K, _L), 1)
  for rows_ref, tgt_ref in ((srow_ref, stt_ref), (erow_ref, ett_ref)):
    x = rows_ref[...]
    m = jnp.max(x, axis=1, keepdims=True)
    lse = m + jnp.log(jnp.sum(jnp.exp(x - m), axis=1, keepdims=True))
    picked = jnp.sum(jnp.where(jl == tgt_ref[...], x, 0.0), axis=1,
                     keepdims=True)
    loss = loss + jnp.sum(lse - picked) * (1.0 / (_B * _K))

  out_ref[...] = jnp.reshape(loss, (1, 1))


def kernel(agg_logits, sel_logits, where_num_logits, where_col_logits,
           where_op_logits, where_start_logits, where_end_logits,
           agg_target, sel_target, where_num_target, where_col_target,
           where_op_target, where_start_target, where_end_target):
  colt = where_col_target.astype(jnp.int32)
  idx = (jnp.arange(_B, dtype=jnp.int32)[:, None] * _C + colt).reshape(-1)
  srows, erows = _sc_gather_rows(
      where_start_logits.reshape(_B * _C, _L),
      where_end_logits.reshape(_B * _C, _L), idx)

  out = pl.pallas_call(
      _tc_loss_body,
      out_shape=jax.ShapeDtypeStruct((1, 1), jnp.float32),
  )(
      sel_logits,
      agg_logits.reshape(_B, _C * _A),
      where_num_logits,
      where_col_logits,
      where_op_logits.reshape(_B, _C * _O),
      srows,
      erows,
      sel_target.astype(jnp.int32).reshape(_B, 1),
      agg_target.astype(jnp.int32).reshape(_B, 1),
      where_num_target.astype(jnp.int32).reshape(_B, 1),
      colt,
      where_op_target.astype(jnp.int32),
      where_start_target.astype(jnp.int32).reshape(_B * _K, 1),
      where_end_target.astype(jnp.int32).reshape(_B * _K, 1),
  )
  return out[0, 0]
